# baseline (speedup 1.0000x reference)
import functools
import math

import jax
import jax.numpy as jnp
from jax.experimental import pallas as pl
from jax.experimental.pallas import tpu as pltpu

B, N, K, H = 4, 2048, 16, 256
BN = 256

_INV_SQRT2 = 1.0 / math.sqrt(2.0)


def _gelu_exact(x):
    return 0.5 * x * (1.0 + jax.lax.erf(x * _INV_SQRT2))


def _dot(a, b, precision=None):
    return jax.lax.dot_general(a, b, (((1,), (0,)), ((), ())),
                               precision=precision,
                               preferred_element_type=jnp.float32)


def _fused_kernel(points_ref, pointsT_ref,
                  cw1_ref, cb1_ref, cw2_ref, cb2_ref,
                  nw1_ref, nb1_ref, nw2_ref, nb2_ref,
                  ow1_ref, ob1_ref, ow2_ref, ob2_ref,
                  out_ref):
    j = pl.program_id(1)

    rows = points_ref[0, pl.ds(j * BN, BN), :]
    rows_xyz = rows[:, 0:3]
    rows_nrm = rows[:, 3:6]
    xyzT = pointsT_ref[0, 0:3, :]

    sq_lane = jnp.sum(xyzT * xyzT, axis=0, keepdims=True)
    sq_rows = jnp.sum(rows_xyz * rows_xyz, axis=1, keepdims=True)
    cross = _dot(rows_xyz, xyzT)
    d2 = sq_rows + sq_lane - 2.0 * cross
    dist = jnp.sqrt(jnp.maximum(d2, 0.0))

    col = jax.lax.broadcasted_iota(jnp.int32, (BN, N), 1)
    big = jnp.int32(N)

    cat6 = points_ref[0, :, :]
    cat6_hi = cat6.astype(jnp.bfloat16).astype(jnp.float32)
    cat6_lo = cat6 - cat6_hi
    cat12 = jnp.concatenate([cat6_hi, cat6_lo], axis=1)

    coord_feat = None
    normal_feat = None
    for i in range(K + 1):
        m = jnp.min(dist, axis=1, keepdims=True)
        cand = jnp.where(dist == m, col, big)
        idx = jnp.min(cand, axis=1, keepdims=True)
        sel = cand == idx
        if i > 0:
            g12 = _dot(sel.astype(jnp.float32), cat12)
            g6 = g12[:, 0:6] + g12[:, 6:12]
            nbr_xyz = g6[:, 0:3]
            nbr_nrm = g6[:, 3:6]

            rel = nbr_xyz - rows_xyz
            h = _gelu_exact(_dot(rel, cw1_ref[...]) + cb1_ref[...])
            ch = _dot(h, cw2_ref[...]) + cb2_ref[...]
            coord_feat = ch if coord_feat is None else jnp.maximum(coord_feat, ch)

            dotc = jnp.sum(nbr_nrm * rows_nrm, axis=1, keepdims=True)
            variation = jnp.abs(1.0 - dotc)
            nin = jnp.concatenate([nbr_nrm, variation], axis=1)
            hn = _gelu_exact(_dot(nin, nw1_ref[...]) + nb1_ref[...])
            nh = _dot(hn, nw2_ref[...]) + nb2_ref[...]
            normal_feat = nh if normal_feat is None else jnp.maximum(normal_feat, nh)
        if i < K:
            dist = jnp.where(sel, jnp.float32(jnp.inf), dist)

    fused = jnp.concatenate([coord_feat, normal_feat], axis=1)
    ho = _gelu_exact(_dot(fused, ow1_ref[...]) + ob1_ref[...])
    out = _dot(ho, ow2_ref[...]) + ob2_ref[...]
    out_ref[0, :, :] = out


@jax.jit
def kernel(points, cw1, cb1, cw2, cb2, nw1, nb1, nw2, nb2, ow1, ob1, ow2, ob2):
    pointsT = jnp.transpose(points, (0, 2, 1))

    def full(a):
        return pl.BlockSpec(a.shape, lambda b, j: (0,) * a.ndim)

    biases = [cb1.reshape(1, H), cb2.reshape(1, H), nb1.reshape(1, H),
              nb2.reshape(1, H), ob1.reshape(1, H), ob2.reshape(1, H)]
    cb1r, cb2r, nb1r, nb2r, ob1r, ob2r = biases

    grid = (B, N // BN)
    out = pl.pallas_call(
        _fused_kernel,
        grid=grid,
        in_specs=[
            pl.BlockSpec((1, N, 6), lambda b, j: (b, 0, 0)),
            pl.BlockSpec((1, 6, N), lambda b, j: (b, 0, 0)),
            full(cw1), full(cb1r), full(cw2), full(cb2r),
            full(nw1), full(nb1r), full(nw2), full(nb2r),
            full(ow1), full(ob1r), full(ow2), full(ob2r),
        ],
        out_specs=pl.BlockSpec((1, BN, H), lambda b, j: (b, j, 0)),
        out_shape=jax.ShapeDtypeStruct((B, N, H), jnp.float32),
        compiler_params=pltpu.CompilerParams(
            dimension_semantics=("parallel", "parallel")),
    )(points, pointsT, cw1, cb1r, cw2, cb2r, nw1, nb1r, nw2, nb2r,
      ow1, ob1r, ow2, ob2r)
    return out


# speedup vs baseline: 1.4538x; 1.4538x over previous
import functools
import math

import jax
import jax.numpy as jnp
from jax.experimental import pallas as pl
from jax.experimental.pallas import tpu as pltpu

B, N, K, H = 4, 2048, 16, 256
BN = 256

_INV_SQRT2 = 1.0 / math.sqrt(2.0)


def _gelu_exact(x):
    return 0.5 * x * (1.0 + jax.lax.erf(x * _INV_SQRT2))


def _dot(a, b, precision=None):
    return jax.lax.dot_general(a, b, (((1,), (0,)), ((), ())),
                               precision=precision,
                               preferred_element_type=jnp.float32)


def _fused_kernel(points_ref, pointsT_ref,
                  cw1_ref, cb1_ref, cw2_ref, cb2_ref,
                  nw1_ref, nb1_ref, nw2_ref, nb2_ref,
                  ow1_ref, ob1_ref, ow2_ref, ob2_ref,
                  out_ref):
    j = pl.program_id(1)

    rows = points_ref[0, pl.ds(j * BN, BN), :]
    rows_xyz = rows[:, 0:3]
    rows_nrm = rows[:, 3:6]
    xyzT = pointsT_ref[0, 0:3, :]
    nrmT = pointsT_ref[0, 3:6, :]

    sq_lane = jnp.sum(xyzT * xyzT, axis=0, keepdims=True)
    sq_rows = jnp.sum(rows_xyz * rows_xyz, axis=1, keepdims=True)
    cross = _dot(rows_xyz, xyzT)
    d2 = sq_rows + sq_lane - 2.0 * cross
    dist = jnp.sqrt(jnp.maximum(d2, 0.0))

    col = jax.lax.broadcasted_iota(jnp.int32, (BN, N), 1).astype(jnp.float32)
    big = jnp.float32(N)

    cat6 = points_ref[0, :, :]
    cat6_hi = cat6.astype(jnp.bfloat16).astype(jnp.float32)
    cat6_lo = cat6 - cat6_hi
    cat12 = jnp.concatenate([cat6_hi, cat6_lo], axis=1)

    nbr = []
    for i in range(K + 1):
        m = jnp.min(dist, axis=1, keepdims=True)
        cand = jnp.where(dist == m, col, big)
        idx = jnp.min(cand, axis=1, keepdims=True)
        sel = cand == idx
        if i > 0:
            g12 = _dot(sel.astype(jnp.float32), cat12)
            nbr.append(g12[:, 0:6] + g12[:, 6:12])
        dist = jnp.where(sel, jnp.float32(jnp.inf), dist)

    nbrs = jnp.stack(nbr, axis=0)
    nbr_xyz = nbrs[:, :, 0:3]
    nbr_nrm = nbrs[:, :, 3:6]

    rel = (nbr_xyz - rows_xyz[None, :, :]).reshape(K * BN, 3)
    h = _gelu_exact(_dot(rel, cw1_ref[...]) + cb1_ref[...])
    ch = _dot(h, cw2_ref[...]) + cb2_ref[...]
    coord_feat = jnp.max(ch.reshape(K, BN, H), axis=0)

    dotc = jnp.sum(nbr_nrm * rows_nrm[None, :, :], axis=2, keepdims=True)
    variation = jnp.abs(1.0 - dotc)
    nin = jnp.concatenate([nbr_nrm, variation], axis=2).reshape(K * BN, 4)
    hn = _gelu_exact(_dot(nin, nw1_ref[...]) + nb1_ref[...])
    nh = _dot(hn, nw2_ref[...]) + nb2_ref[...]
    normal_feat = jnp.max(nh.reshape(K, BN, H), axis=0)

    fused = jnp.concatenate([coord_feat, normal_feat], axis=1)
    ho = _gelu_exact(_dot(fused, ow1_ref[...]) + ob1_ref[...])
    out = _dot(ho, ow2_ref[...]) + ob2_ref[...]
    out_ref[0, :, :] = out


@jax.jit
def kernel(points, cw1, cb1, cw2, cb2, nw1, nb1, nw2, nb2, ow1, ob1, ow2, ob2):
    pointsT = jnp.transpose(points, (0, 2, 1))

    def full(a):
        return pl.BlockSpec(a.shape, lambda b, j: (0,) * a.ndim)

    biases = [cb1.reshape(1, H), cb2.reshape(1, H), nb1.reshape(1, H),
              nb2.reshape(1, H), ob1.reshape(1, H), ob2.reshape(1, H)]
    cb1r, cb2r, nb1r, nb2r, ob1r, ob2r = biases

    grid = (B, N // BN)
    out = pl.pallas_call(
        _fused_kernel,
        grid=grid,
        in_specs=[
            pl.BlockSpec((1, N, 6), lambda b, j: (b, 0, 0)),
            pl.BlockSpec((1, 6, N), lambda b, j: (b, 0, 0)),
            full(cw1), full(cb1r), full(cw2), full(cb2r),
            full(nw1), full(nb1r), full(nw2), full(nb2r),
            full(ow1), full(ob1r), full(ow2), full(ob2r),
        ],
        out_specs=pl.BlockSpec((1, BN, H), lambda b, j: (b, j, 0)),
        out_shape=jax.ShapeDtypeStruct((B, N, H), jnp.float32),
        compiler_params=pltpu.CompilerParams(
            dimension_semantics=("parallel", "parallel")),
    )(points, pointsT, cw1, cb1r, cw2, cb2r, nw1, nb1r, nw2, nb2r,
      ow1, ob1r, ow2, ob2r)
    return out


# speedup vs baseline: 1.4540x; 1.0002x over previous
import math

import jax
import jax.numpy as jnp
from jax.experimental import pallas as pl
from jax.experimental.pallas import tpu as pltpu

B, N, K, H = 4, 2048, 16, 256
BN = 256

_INV_SQRT2 = 1.0 / math.sqrt(2.0)


def _gelu_exact(x):
    return 0.5 * x * (1.0 + jax.lax.erf(x * _INV_SQRT2))


def _dot(a, b, precision=None):
    return jax.lax.dot_general(a, b, (((1,), (0,)), ((), ())),
                               precision=precision,
                               preferred_element_type=jnp.float32)


def _fused_kernel(points_ref, pointsT_ref,
                  cw1_ref, cb1_ref, cw2_ref, cb2_ref,
                  nw1_ref, nb1_ref, nw2_ref, nb2_ref,
                  ow1_ref, ob1_ref, ow2_ref, ob2_ref,
                  out_ref):
    j = pl.program_id(1)

    rows = points_ref[0, pl.ds(j * BN, BN), :]
    rows_xyz = rows[:, 0:3]
    rows_nrm = rows[:, 3:6]
    xyzT = pointsT_ref[0, 0:3, :]

    sq_lane = jnp.sum(xyzT * xyzT, axis=0, keepdims=True)
    sq_rows = jnp.sum(rows_xyz * rows_xyz, axis=1, keepdims=True)
    cross = _dot(rows_xyz, xyzT)
    d2 = sq_rows + sq_lane - 2.0 * cross
    dist = jnp.sqrt(jnp.maximum(d2, 0.0))

    col = jax.lax.broadcasted_iota(jnp.int32, (BN, N), 1).astype(jnp.float32)
    big = jnp.float32(N)

    cat6 = points_ref[0, :, :]
    cat6_hi = cat6.astype(jnp.bfloat16).astype(jnp.float32)
    cat6_lo = cat6 - cat6_hi
    cat12 = jnp.concatenate([cat6_hi, cat6_lo], axis=1)

    nbr = []
    for i in range(K + 1):
        m = jnp.min(dist, axis=1, keepdims=True)
        cand = jnp.where(dist == m, col, big)
        idx = jnp.min(cand, axis=1, keepdims=True)
        sel = cand == idx
        if i > 0:
            g12 = _dot(sel.astype(jnp.float32), cat12)
            nbr.append(g12[:, 0:6] + g12[:, 6:12])
        dist = jnp.where(sel, jnp.float32(jnp.inf), dist)

    nbrs = jnp.stack(nbr, axis=0)
    nbr_xyz = nbrs[:, :, 0:3]
    nbr_nrm = nbrs[:, :, 3:6]

    rel = (nbr_xyz - rows_xyz[None, :, :]).reshape(K * BN, 3)
    h = _gelu_exact(_dot(rel, cw1_ref[...]) + cb1_ref[...])
    ch = _dot(h, cw2_ref[...]) + cb2_ref[...]
    coord_feat = jnp.max(ch.reshape(K, BN, H), axis=0)

    dotc = jnp.sum(nbr_nrm * rows_nrm[None, :, :], axis=2, keepdims=True)
    variation = jnp.abs(1.0 - dotc)
    nin = jnp.concatenate([nbr_nrm, variation], axis=2).reshape(K * BN, 4)
    hn = _gelu_exact(_dot(nin, nw1_ref[...]) + nb1_ref[...])
    nh = _dot(hn, nw2_ref[...]) + nb2_ref[...]
    normal_feat = jnp.max(nh.reshape(K, BN, H), axis=0)

    fused = jnp.concatenate([coord_feat, normal_feat], axis=1)
    ho = _gelu_exact(_dot(fused, ow1_ref[...]) + ob1_ref[...])
    out = _dot(ho, ow2_ref[...]) + ob2_ref[...]
    out_ref[0, :, :] = out


@jax.jit
def kernel(points, cw1, cb1, cw2, cb2, nw1, nb1, nw2, nb2, ow1, ob1, ow2, ob2):
    pointsT = jnp.transpose(points, (0, 2, 1))

    def full(a):
        return pl.BlockSpec(a.shape, lambda b, j: (0,) * a.ndim)

    biases = [cb1.reshape(1, H), cb2.reshape(1, H), nb1.reshape(1, H),
              nb2.reshape(1, H), ob1.reshape(1, H), ob2.reshape(1, H)]
    cb1r, cb2r, nb1r, nb2r, ob1r, ob2r = biases

    grid = (B, N // BN)
    out = pl.pallas_call(
        _fused_kernel,
        grid=grid,
        in_specs=[
            pl.BlockSpec((1, N, 6), lambda b, j: (b, 0, 0)),
            pl.BlockSpec((1, 6, N), lambda b, j: (b, 0, 0)),
            full(cw1), full(cb1r), full(cw2), full(cb2r),
            full(nw1), full(nb1r), full(nw2), full(nb2r),
            full(ow1), full(ob1r), full(ow2), full(ob2r),
        ],
        out_specs=pl.BlockSpec((1, BN, H), lambda b, j: (b, j, 0)),
        out_shape=jax.ShapeDtypeStruct((B, N, H), jnp.float32),
        compiler_params=pltpu.CompilerParams(
            dimension_semantics=("parallel", "parallel")),
    )(points, pointsT, cw1, cb1r, cw2, cb2r, nw1, nb1r, nw2, nb2r,
      ow1, ob1r, ow2, ob2r)
    return out
